# Initial kernel scaffold; baseline (speedup 1.0000x reference)
import jax
import jax.numpy as jnp
from jax.experimental import pallas as pl


def kernel(x_sotu, x_taxon, edge_label_index):
    raise NotImplementedError("write your pallas kernel here")


# speedup vs baseline: 1.2125x; 1.2125x over previous
import functools

import jax
import jax.numpy as jnp
from jax import lax
from jax.experimental import pallas as pl
from jax.experimental.pallas import tpu as pltpu
from jax.experimental.pallas import tpu_sc as plsc

B = 320000
D = 128
NW = 32
E_PER_W = B // NW
C = 400
N_CHUNKS = E_PER_W // C
GROUPS = C // 16

_mesh = plsc.VectorSubcoreMesh(core_axis_name="c", subcore_axis_name="s")


@functools.partial(
    pl.kernel,
    out_type=jax.ShapeDtypeStruct((B,), jnp.float32),
    mesh=_mesh,
    scratch_types=[
        pltpu.VMEM((C,), jnp.int32),
        pltpu.VMEM((C,), jnp.int32),
        pltpu.VMEM((C, D), jnp.float32),
        pltpu.VMEM((C, D), jnp.float32),
        pltpu.VMEM((C,), jnp.float32),
        pltpu.SemaphoreType.DMA,
    ],
    compiler_params=pltpu.CompilerParams(needs_layout_passes=False),
)
def _sc_kernel(x_sotu_hbm, x_taxon_hbm, idx0_hbm, idx1_hbm, out_hbm,
               idx0_v, idx1_v, rows0_v, rows1_v, out_v, sem):
    wid = lax.axis_index("s") * 2 + lax.axis_index("c")
    lane = lax.iota(jnp.int32, 16)

    def chunk_body(it, _):
        base = wid * E_PER_W + it * C
        pltpu.sync_copy(idx0_hbm.at[pl.ds(base, C)], idx0_v)
        pltpu.sync_copy(idx1_hbm.at[pl.ds(base, C)], idx1_v)
        cp0 = pltpu.async_copy(x_sotu_hbm.at[idx0_v], rows0_v, sem)
        cp1 = pltpu.async_copy(x_taxon_hbm.at[idx1_v], rows1_v, sem)
        cp0.wait()
        cp1.wait()

        def group_body(g, _):
            row_idx = lane + g * 16
            col = jnp.zeros((16,), jnp.int32)
            one = jnp.ones((16,), jnp.int32)
            acc = jnp.zeros((16,), jnp.float32)
            for d in range(D):
                a = plsc.load_gather(rows0_v, [row_idx, col])
                b = plsc.load_gather(rows1_v, [row_idx, col])
                acc = acc + a * b
                col = col + one
            out_v[pl.ds(g * 16, 16)] = acc
            return 0

        lax.fori_loop(0, GROUPS, group_body, 0)
        pltpu.sync_copy(out_v, out_hbm.at[pl.ds(base, C)])
        return 0

    lax.fori_loop(0, N_CHUNKS, chunk_body, 0)


def kernel(x_sotu, x_taxon, edge_label_index):
    idx0 = edge_label_index[0]
    idx1 = edge_label_index[1]
    return _sc_kernel(x_sotu, x_taxon, idx0, idx1)
